# Initial kernel scaffold; baseline (speedup 1.0000x reference)
import jax
import jax.numpy as jnp
from jax.experimental import pallas as pl


def kernel(x, edge_index, edge_attr, batch, av_w, av_b, gin_eps, l1_w1, l1_b1, l1_w2, l1_b2, l2_w1, l2_b1, l2_w2, l2_b2, l3_w1, l3_b1, l3_w2, l3_b2, l4_w1, l4_b1, l4_w2, l4_b2, fc_w, fc_b):
    raise NotImplementedError("write your pallas kernel here")


# speedup vs baseline: 1.8915x; 1.8915x over previous
import functools

import jax
import jax.numpy as jnp
from jax import lax
from jax.experimental import pallas as pl
from jax.experimental.pallas import tpu as pltpu
from jax.experimental.pallas import tpu_sc as plsc

N = 10000
E = 160000
C = 128
NP = 10240
CH = 128
NWORK = 32
PER_TILE = 5120
NCHUNK = PER_TILE // CH
EP = NWORK * PER_TILE
RSLICE = NP // 16
SCALE = float(1.0 / (1.0 + 1e-5) ** 0.5)
BN = 1024
BE = 2048

_f32 = jnp.float32


def _wid_base():
    cid = lax.axis_index("c")
    sid = lax.axis_index("s")
    wid = cid * 16 + sid
    return cid, sid, wid * PER_TILE


def _gin_sc_body(gf, srcp, dstp, parts, si, di, rows, acc, sem):
    cid, sid, base0 = _wid_base()
    r0 = sid * RSLICE

    def fill(i, carry):
        zv = jnp.zeros((16,), _f32)
        for k in range(C // 16):
            rows[i, pl.ds(k * 16, 16)] = zv
        return carry

    lax.fori_loop(0, CH, fill, 0)
    for k in range(RSLICE // CH):
        pltpu.sync_copy(rows, acc.at[pl.ds(r0 + k * CH, CH)])
    plsc.subcore_barrier()

    def chunk(j, carry):
        base = base0 + j * CH
        pltpu.sync_copy(srcp.at[pl.ds(base, CH)], si)
        pltpu.sync_copy(dstp.at[pl.ds(base, CH)], di)
        pltpu.async_copy(gf.at[si], rows, sem).wait()
        pltpu.sync_copy(rows, acc.at[di], add=True)
        return carry

    lax.fori_loop(0, NCHUNK, chunk, 0)
    plsc.subcore_barrier()
    out0 = cid * NP + r0
    for k in range(RSLICE // CH):
        pltpu.sync_copy(acc.at[pl.ds(r0 + k * CH, CH)], rows)
        pltpu.sync_copy(rows, parts.at[pl.ds(out0 + k * CH, CH)])


def _cnt_sc_body(dstp, cnts, di, ones, w16, cacc):
    cid, sid, base0 = _wid_base()
    r0 = sid * RSLICE

    def fill(i, carry):
        ov = jnp.full((16,), 1.0, _f32)
        zv = jnp.zeros((16,), _f32)
        for k in range(C // 16):
            ones[i, pl.ds(k * 16, 16)] = ov
            w16[i, pl.ds(k * 16, 16)] = zv
        return carry

    lax.fori_loop(0, CH, fill, 0)
    for k in range(RSLICE // CH):
        pltpu.sync_copy(w16, cacc.at[pl.ds(r0 + k * CH, CH)])
    plsc.subcore_barrier()

    def chunk(j, carry):
        base = base0 + j * CH
        pltpu.sync_copy(dstp.at[pl.ds(base, CH)], di)
        pltpu.sync_copy(ones, cacc.at[di], add=True)
        return carry

    lax.fori_loop(0, NCHUNK, chunk, 0)
    plsc.subcore_barrier()
    out0 = cid * NP + r0
    for k in range(RSLICE // CH):
        pltpu.sync_copy(cacc.at[pl.ds(r0 + k * CH, CH)], w16)
        pltpu.sync_copy(w16, cnts.at[pl.ds(out0 + k * CH, CH)])


def _gat_sc_body(u_hbm, v_hbm, srcp, dstp, ug, vg, si, di, ub, vb, s1, s2):
    _, _, base0 = _wid_base()

    def chunk(j, carry):
        base = base0 + j * CH
        pltpu.sync_copy(dstp.at[pl.ds(base, CH)], di)
        pltpu.sync_copy(srcp.at[pl.ds(base, CH)], si)
        c1 = pltpu.async_copy(u_hbm.at[di], ub, s1)
        c2 = pltpu.async_copy(v_hbm.at[si], vb, s2)
        c1.wait()
        c2.wait()
        pltpu.sync_copy(ub, ug.at[pl.ds(base, CH)])
        pltpu.sync_copy(vb, vg.at[pl.ds(base, CH)])
        return carry

    lax.fori_loop(0, NCHUNK, chunk, 0)


def _sct_sc_body(h2, dstp, parts, di, rows, acc):
    cid, sid, base0 = _wid_base()
    r0 = sid * RSLICE

    def fill(i, carry):
        zv = jnp.zeros((16,), _f32)
        for k in range(C // 16):
            rows[i, pl.ds(k * 16, 16)] = zv
        return carry

    lax.fori_loop(0, CH, fill, 0)
    for k in range(RSLICE // CH):
        pltpu.sync_copy(rows, acc.at[pl.ds(r0 + k * CH, CH)])
    plsc.subcore_barrier()

    def chunk(j, carry):
        base = base0 + j * CH
        pltpu.sync_copy(dstp.at[pl.ds(base, CH)], di)
        pltpu.sync_copy(h2.at[pl.ds(base, CH)], rows)
        pltpu.sync_copy(rows, acc.at[di], add=True)
        return carry

    lax.fori_loop(0, NCHUNK, chunk, 0)
    plsc.subcore_barrier()
    out0 = cid * NP + r0
    for k in range(RSLICE // CH):
        pltpu.sync_copy(acc.at[pl.ds(r0 + k * CH, CH)], rows)
        pltpu.sync_copy(rows, parts.at[pl.ds(out0 + k * CH, CH)])


@functools.lru_cache(maxsize=None)
def _sc_kernels():
    mesh = plsc.VectorSubcoreMesh(core_axis_name="c", subcore_axis_name="s")
    gin = pl.kernel(
        _gin_sc_body,
        mesh=mesh,
        out_type=jax.ShapeDtypeStruct((2 * NP, C), _f32),
        scratch_types=[pltpu.VMEM((CH,), jnp.int32),
                       pltpu.VMEM((CH,), jnp.int32),
                       pltpu.VMEM((CH, C), _f32),
                       pltpu.VMEM_SHARED((NP, C), _f32),
                       pltpu.SemaphoreType.DMA],
    )
    cntk = pl.kernel(
        _cnt_sc_body,
        mesh=mesh,
        out_type=jax.ShapeDtypeStruct((2 * NP, C), _f32),
        scratch_types=[pltpu.VMEM((CH,), jnp.int32),
                       pltpu.VMEM((CH, C), _f32), pltpu.VMEM((CH, C), _f32),
                       pltpu.VMEM_SHARED((NP, C), _f32)],
    )
    gather2 = pl.kernel(
        _gat_sc_body,
        mesh=mesh,
        out_type=[jax.ShapeDtypeStruct((EP, C), _f32),
                  jax.ShapeDtypeStruct((EP, C), _f32)],
        scratch_types=[pltpu.VMEM((CH,), jnp.int32),
                       pltpu.VMEM((CH,), jnp.int32),
                       pltpu.VMEM((CH, C), _f32), pltpu.VMEM((CH, C), _f32),
                       pltpu.SemaphoreType.DMA, pltpu.SemaphoreType.DMA],
    )
    scatter = pl.kernel(
        _sct_sc_body,
        mesh=mesh,
        out_type=jax.ShapeDtypeStruct((2 * NP, C), _f32),
        scratch_types=[pltpu.VMEM((CH,), jnp.int32), pltpu.VMEM((CH, C), _f32),
                       pltpu.VMEM_SHARED((NP, C), _f32)],
    )
    return gin, cntk, gather2, scatter


def _sc_gin(*a):
    return _sc_kernels()[0](*a)


def _sc_cnt(*a):
    return _sc_kernels()[1](*a)


def _sc_gather2(*a):
    return _sc_kernels()[2](*a)


def _sc_scatter(*a):
    return _sc_kernels()[3](*a)


def _dot(a, b):
    return jnp.dot(a, b, preferred_element_type=_f32)


def _fuse_body(x2, w, b, o):
    o[...] = _dot(x2[...], w[...]) + b[...]


def _gin_body(eps, gf, p0, p1, w1, b1, w2, b2, am, bm, b1n, g_o, u_o, v_o):
    z = gf[...] * (1.0 + eps[0, 0]) + p0[...] + p1[...]
    h = jnp.maximum((_dot(z, w1[...]) + b1[...]) * SCALE, 0.0)
    h = jnp.maximum((_dot(h, w2[...]) + b2[...]) * SCALE, 0.0)
    g = h * SCALE
    g_o[...] = g
    u_o[...] = (_dot(g, am[...]) + b1n[...]) * SCALE
    v_o[...] = _dot(g, bm[...]) * SCALE


def _mid_body(ug, vg, w2, b2, o):
    h1 = jnp.maximum(ug[...] + vg[...], 0.0)
    o[...] = jnp.maximum((_dot(h1, w2[...]) + b2[...]) * SCALE, 0.0)


def _post_body(p0, p1, c0, c1, gp, am, bm, b1n, g_o, u_o, v_o):
    cnt = jnp.maximum(c0[:, :1] + c1[:, :1], 1.0)
    g = jnp.maximum(((p0[...] + p1[...]) / cnt + gp[...]) * SCALE, 0.0)
    g_o[...] = g
    u_o[...] = (_dot(g, am[...]) + b1n[...]) * SCALE
    v_o[...] = _dot(g, bm[...]) * SCALE


def _fin_body(p0, p1, c0, c1, gp, fw, fb, o):
    cnt = jnp.maximum(c0[:, :1] + c1[:, :1], 1.0)
    g4 = (p0[...] + p1[...]) / cnt + gp[...]
    o[...] = _dot(g4, fw[...]) + fb[...]


def _row_spec(bn, cols):
    return pl.BlockSpec((bn, cols), lambda i: (i, 0))


def _p1_spec(bn, cols, off):
    return pl.BlockSpec((bn, cols), lambda i: (i + off, 0))


def _w_spec(r, c=C):
    return pl.BlockSpec((r, c), lambda i: (0, 0))


_NODE_GRID = NP // BN
_EDGE_GRID = EP // BE


def _tc_fuse(x2p, av_w, av_b):
    return pl.pallas_call(
        _fuse_body,
        grid=(_NODE_GRID,),
        in_specs=[_row_spec(BN, 2 * C), _w_spec(2 * C), _w_spec(1)],
        out_specs=_row_spec(BN, C),
        out_shape=jax.ShapeDtypeStruct((NP, C), _f32),
    )(x2p, av_w, av_b.reshape(1, C))


_NODE_OUT3 = [jax.ShapeDtypeStruct((NP, C), _f32)] * 3


def _tc_gin(eps, gf, parts, w1, b1, w2, b2, am, bm, b1n):
    return pl.pallas_call(
        _gin_body,
        grid=(_NODE_GRID,),
        in_specs=[pl.BlockSpec((1, 1), lambda i: (0, 0)),
                  _row_spec(BN, C), _row_spec(BN, C),
                  _p1_spec(BN, C, _NODE_GRID),
                  _w_spec(C), _w_spec(1), _w_spec(C), _w_spec(1),
                  _w_spec(C), _w_spec(C), _w_spec(1)],
        out_specs=[_row_spec(BN, C)] * 3,
        out_shape=_NODE_OUT3,
    )(eps.reshape(1, 1), gf, parts, parts, w1, b1.reshape(1, C),
      w2, b2.reshape(1, C), am, bm, b1n.reshape(1, C))


def _tc_mid(ug, vg, w2, b2):
    return pl.pallas_call(
        _mid_body,
        grid=(_EDGE_GRID,),
        in_specs=[_row_spec(BE, C), _row_spec(BE, C), _w_spec(C), _w_spec(1)],
        out_specs=_row_spec(BE, C),
        out_shape=jax.ShapeDtypeStruct((EP, C), _f32),
    )(ug, vg, w2, b2.reshape(1, C))


def _tc_post(parts, cnts, gp, am, bm, b1n):
    return pl.pallas_call(
        _post_body,
        grid=(_NODE_GRID,),
        in_specs=[_row_spec(BN, C), _p1_spec(BN, C, _NODE_GRID),
                  _row_spec(BN, C), _p1_spec(BN, C, _NODE_GRID),
                  _row_spec(BN, C),
                  _w_spec(C), _w_spec(C), _w_spec(1)],
        out_specs=[_row_spec(BN, C)] * 3,
        out_shape=_NODE_OUT3,
    )(parts, parts, cnts, cnts, gp, am, bm, b1n.reshape(1, C))


def _tc_fin(parts, cnts, gp, fw, fb):
    return pl.pallas_call(
        _fin_body,
        grid=(_NODE_GRID,),
        in_specs=[_row_spec(BN, C), _p1_spec(BN, C, _NODE_GRID),
                  _row_spec(BN, C), _p1_spec(BN, C, _NODE_GRID),
                  _row_spec(BN, C), _w_spec(C), _w_spec(1)],
        out_specs=_row_spec(BN, C),
        out_shape=jax.ShapeDtypeStruct((NP, C), _f32),
    )(parts, parts, cnts, cnts, gp, fw, fb)


def kernel(x, edge_index, edge_attr, batch, av_w, av_b, gin_eps,
           l1_w1, l1_b1, l1_w2, l1_b2,
           l2_w1, l2_b1, l2_w2, l2_b2,
           l3_w1, l3_b1, l3_w2, l3_b2,
           l4_w1, l4_b1, l4_w2, l4_b2,
           fc_w, fc_b):
    x2 = x.reshape(N, 2 * C)
    x2p = jnp.pad(x2, ((0, NP - N), (0, 0)))
    srcp = jnp.pad(edge_index[0], (0, EP - E), constant_values=N)
    dstp = jnp.pad(edge_index[1], (0, EP - E), constant_values=N)
    def split_w1(w1):
        return w1[:C] - w1[C:], w1[C:]

    am2, bm2 = split_w1(l2_w1)
    am3, bm3 = split_w1(l3_w1)
    am4, bm4 = split_w1(l4_w1)
    fw = jnp.zeros((C, C), _f32).at[:, :2].set(fc_w)
    fb = jnp.zeros((1, C), _f32).at[0, :2].set(fc_b)

    gf = _tc_fuse(x2p, av_w, av_b)
    cnts = _sc_cnt(dstp)
    parts0 = _sc_gin(gf, srcp, dstp)
    g1, u2, v2 = _tc_gin(gin_eps, gf, parts0, l1_w1, l1_b1, l1_w2, l1_b2,
                         am2, bm2, l2_b1)

    ug, vg = _sc_gather2(u2, v2, srcp, dstp)
    h2 = _tc_mid(ug, vg, l2_w2, l2_b2)
    parts2 = _sc_scatter(h2, dstp)
    g2, u3, v3 = _tc_post(parts2, cnts, g1, am3, bm3, l3_b1)

    ug, vg = _sc_gather2(u3, v3, srcp, dstp)
    h3 = _tc_mid(ug, vg, l3_w2, l3_b2)
    parts3 = _sc_scatter(h3, dstp)
    g3, u4, v4 = _tc_post(parts3, cnts, g2, am4, bm4, l4_b1)

    ug, vg = _sc_gather2(u4, v4, srcp, dstp)
    h4 = _tc_mid(ug, vg, l4_w2, l4_b2)
    parts4 = _sc_scatter(h4, dstp)
    outp = _tc_fin(parts4, cnts, g3, fw, fb)
    return outp[:N, :2]


# speedup vs baseline: 2.1054x; 1.1131x over previous
import functools

import jax
import jax.numpy as jnp
from jax import lax
from jax.experimental import pallas as pl
from jax.experimental.pallas import tpu as pltpu
from jax.experimental.pallas import tpu_sc as plsc

N = 10000
E = 160000
C = 128
NP = 10240
CH = 128
NWORK = 32
PER_TILE = 5120
NCHUNK = PER_TILE // CH
EP = NWORK * PER_TILE
RSLICE = NP // 16
SCALE = float(1.0 / (1.0 + 1e-5) ** 0.5)
BN = 1024
BE = 2048

_f32 = jnp.float32


def _wid_base():
    cid = lax.axis_index("c")
    sid = lax.axis_index("s")
    wid = cid * 16 + sid
    return cid, sid, wid * PER_TILE


def _gin_sc_body(gf, srcp, dstp, parts, si, di, rows, acc, sem):
    cid, sid, base0 = _wid_base()
    r0 = sid * RSLICE

    def fill(i, carry):
        zv = jnp.zeros((16,), _f32)
        for k in range(C // 16):
            rows[i, pl.ds(k * 16, 16)] = zv
        return carry

    lax.fori_loop(0, CH, fill, 0)
    for k in range(RSLICE // CH):
        pltpu.sync_copy(rows, acc.at[pl.ds(r0 + k * CH, CH)])
    plsc.subcore_barrier()

    def chunk(j, carry):
        base = base0 + j * CH
        pltpu.sync_copy(srcp.at[pl.ds(base, CH)], si)
        pltpu.sync_copy(dstp.at[pl.ds(base, CH)], di)
        pltpu.async_copy(gf.at[si], rows, sem).wait()
        pltpu.sync_copy(rows, acc.at[di], add=True)
        return carry

    lax.fori_loop(0, NCHUNK, chunk, 0)
    plsc.subcore_barrier()
    out0 = cid * NP + r0
    for k in range(RSLICE // CH):
        pltpu.sync_copy(acc.at[pl.ds(r0 + k * CH, CH)], rows)
        pltpu.sync_copy(rows, parts.at[pl.ds(out0 + k * CH, CH)])


def _cnt_sc_body(dstp, cnts, di, ones, w16, cacc):
    cid, sid, base0 = _wid_base()
    r0 = sid * RSLICE

    def fill(i, carry):
        ov = jnp.full((16,), 1.0, _f32)
        zv = jnp.zeros((16,), _f32)
        for k in range(C // 16):
            ones[i, pl.ds(k * 16, 16)] = ov
            w16[i, pl.ds(k * 16, 16)] = zv
        return carry

    lax.fori_loop(0, CH, fill, 0)
    for k in range(RSLICE // CH):
        pltpu.sync_copy(w16, cacc.at[pl.ds(r0 + k * CH, CH)])
    plsc.subcore_barrier()

    def chunk(j, carry):
        base = base0 + j * CH
        pltpu.sync_copy(dstp.at[pl.ds(base, CH)], di)
        pltpu.sync_copy(ones, cacc.at[di], add=True)
        return carry

    lax.fori_loop(0, NCHUNK, chunk, 0)
    plsc.subcore_barrier()
    out0 = cid * NP + r0
    for k in range(RSLICE // CH):
        pltpu.sync_copy(cacc.at[pl.ds(r0 + k * CH, CH)], w16)
        pltpu.sync_copy(w16, cnts.at[pl.ds(out0 + k * CH, CH)])


def _gat_sc_body(u_hbm, v_hbm, srcp, dstp, ug, vg,
                 si_all, di_all, ub0, vb0, ub1, vb1, sU0, sV0, sU1, sV1):
    _, _, base0 = _wid_base()
    pltpu.sync_copy(srcp.at[pl.ds(base0, PER_TILE)], si_all)
    pltpu.sync_copy(dstp.at[pl.ds(base0, PER_TILE)], di_all)
    ubs, vbs = (ub0, ub1), (vb0, vb1)
    sUs, sVs = (sU0, sU1), (sV0, sV1)

    def _gat(j, p):
        off = pl.ds(j * CH, CH)
        return (pltpu.make_async_copy(u_hbm.at[di_all.at[off]], ubs[p], sUs[p]),
                pltpu.make_async_copy(v_hbm.at[si_all.at[off]], vbs[p], sVs[p]))

    def _put(j, p):
        hb = pl.ds(base0 + j * CH, CH)
        pltpu.sync_copy(ubs[p], ug.at[hb])
        pltpu.sync_copy(vbs[p], vg.at[hb])

    def start(cs):
        for c in cs:
            c.start()

    def wait(cs):
        for c in cs:
            c.wait()

    start(_gat(0, 0))

    def body(i, carry):
        @pl.when(i > 0)
        def _():
            start(_gat(2 * i, 0))
            wait(_gat(2 * i - 1, 1))
            _put(2 * i - 1, 1)

        start(_gat(2 * i + 1, 1))
        wait(_gat(2 * i, 0))
        _put(2 * i, 0)
        return carry

    lax.fori_loop(0, NCHUNK // 2, body, 0)
    wait(_gat(NCHUNK - 1, 1))
    _put(NCHUNK - 1, 1)


def _sct_sc_body(h2, dstp, parts, di, rows, acc):
    cid, sid, base0 = _wid_base()
    r0 = sid * RSLICE

    def fill(i, carry):
        zv = jnp.zeros((16,), _f32)
        for k in range(C // 16):
            rows[i, pl.ds(k * 16, 16)] = zv
        return carry

    lax.fori_loop(0, CH, fill, 0)
    for k in range(RSLICE // CH):
        pltpu.sync_copy(rows, acc.at[pl.ds(r0 + k * CH, CH)])
    plsc.subcore_barrier()

    def chunk(j, carry):
        base = base0 + j * CH
        pltpu.sync_copy(dstp.at[pl.ds(base, CH)], di)
        pltpu.sync_copy(h2.at[pl.ds(base, CH)], rows)
        pltpu.sync_copy(rows, acc.at[di], add=True)
        return carry

    lax.fori_loop(0, NCHUNK, chunk, 0)
    plsc.subcore_barrier()
    out0 = cid * NP + r0
    for k in range(RSLICE // CH):
        pltpu.sync_copy(acc.at[pl.ds(r0 + k * CH, CH)], rows)
        pltpu.sync_copy(rows, parts.at[pl.ds(out0 + k * CH, CH)])


@functools.lru_cache(maxsize=None)
def _sc_kernels():
    mesh = plsc.VectorSubcoreMesh(core_axis_name="c", subcore_axis_name="s")
    gin = pl.kernel(
        _gin_sc_body,
        mesh=mesh,
        out_type=jax.ShapeDtypeStruct((2 * NP, C), _f32),
        scratch_types=[pltpu.VMEM((CH,), jnp.int32),
                       pltpu.VMEM((CH,), jnp.int32),
                       pltpu.VMEM((CH, C), _f32),
                       pltpu.VMEM_SHARED((NP, C), _f32),
                       pltpu.SemaphoreType.DMA],
    )
    cntk = pl.kernel(
        _cnt_sc_body,
        mesh=mesh,
        out_type=jax.ShapeDtypeStruct((2 * NP, C), _f32),
        scratch_types=[pltpu.VMEM((CH,), jnp.int32),
                       pltpu.VMEM((CH, C), _f32), pltpu.VMEM((CH, C), _f32),
                       pltpu.VMEM_SHARED((NP, C), _f32)],
    )
    gather2 = pl.kernel(
        _gat_sc_body,
        mesh=mesh,
        out_type=[jax.ShapeDtypeStruct((EP, C), _f32),
                  jax.ShapeDtypeStruct((EP, C), _f32)],
        scratch_types=[pltpu.VMEM((PER_TILE,), jnp.int32),
                       pltpu.VMEM((PER_TILE,), jnp.int32),
                       pltpu.VMEM((CH, C), _f32), pltpu.VMEM((CH, C), _f32),
                       pltpu.VMEM((CH, C), _f32), pltpu.VMEM((CH, C), _f32),
                       pltpu.SemaphoreType.DMA, pltpu.SemaphoreType.DMA,
                       pltpu.SemaphoreType.DMA, pltpu.SemaphoreType.DMA],
    )
    scatter = pl.kernel(
        _sct_sc_body,
        mesh=mesh,
        out_type=jax.ShapeDtypeStruct((2 * NP, C), _f32),
        scratch_types=[pltpu.VMEM((CH,), jnp.int32), pltpu.VMEM((CH, C), _f32),
                       pltpu.VMEM_SHARED((NP, C), _f32)],
    )
    return gin, cntk, gather2, scatter


def _sc_gin(*a):
    return _sc_kernels()[0](*a)


def _sc_cnt(*a):
    return _sc_kernels()[1](*a)


def _sc_gather2(*a):
    return _sc_kernels()[2](*a)


def _sc_scatter(*a):
    return _sc_kernels()[3](*a)


def _dot(a, b):
    return jnp.dot(a, b, preferred_element_type=_f32)


def _fuse_body(x2, w, b, o):
    o[...] = _dot(x2[...], w[...]) + b[...]


def _gin_body(eps, gf, p0, p1, w1, b1, w2, b2, am, bm, b1n, g_o, u_o, v_o):
    z = gf[...] * (1.0 + eps[0, 0]) + p0[...] + p1[...]
    h = jnp.maximum((_dot(z, w1[...]) + b1[...]) * SCALE, 0.0)
    h = jnp.maximum((_dot(h, w2[...]) + b2[...]) * SCALE, 0.0)
    g = h * SCALE
    g_o[...] = g
    u_o[...] = (_dot(g, am[...]) + b1n[...]) * SCALE
    v_o[...] = _dot(g, bm[...]) * SCALE


def _mid_body(ug, vg, w2, b2, o):
    h1 = jnp.maximum(ug[...] + vg[...], 0.0)
    o[...] = jnp.maximum((_dot(h1, w2[...]) + b2[...]) * SCALE, 0.0)


def _post_body(p0, p1, c0, c1, gp, am, bm, b1n, g_o, u_o, v_o):
    cnt = jnp.maximum(c0[:, :1] + c1[:, :1], 1.0)
    g = jnp.maximum(((p0[...] + p1[...]) / cnt + gp[...]) * SCALE, 0.0)
    g_o[...] = g
    u_o[...] = (_dot(g, am[...]) + b1n[...]) * SCALE
    v_o[...] = _dot(g, bm[...]) * SCALE


def _fin_body(p0, p1, c0, c1, gp, fw, fb, o):
    cnt = jnp.maximum(c0[:, :1] + c1[:, :1], 1.0)
    g4 = (p0[...] + p1[...]) / cnt + gp[...]
    o[...] = _dot(g4, fw[...]) + fb[...]


def _row_spec(bn, cols):
    return pl.BlockSpec((bn, cols), lambda i: (i, 0))


def _p1_spec(bn, cols, off):
    return pl.BlockSpec((bn, cols), lambda i: (i + off, 0))


def _w_spec(r, c=C):
    return pl.BlockSpec((r, c), lambda i: (0, 0))


_NODE_GRID = NP // BN
_EDGE_GRID = EP // BE


def _tc_fuse(x2p, av_w, av_b):
    return pl.pallas_call(
        _fuse_body,
        grid=(_NODE_GRID,),
        in_specs=[_row_spec(BN, 2 * C), _w_spec(2 * C), _w_spec(1)],
        out_specs=_row_spec(BN, C),
        out_shape=jax.ShapeDtypeStruct((NP, C), _f32),
    )(x2p, av_w, av_b.reshape(1, C))


_NODE_OUT3 = [jax.ShapeDtypeStruct((NP, C), _f32)] * 3


def _tc_gin(eps, gf, parts, w1, b1, w2, b2, am, bm, b1n):
    return pl.pallas_call(
        _gin_body,
        grid=(_NODE_GRID,),
        in_specs=[pl.BlockSpec((1, 1), lambda i: (0, 0)),
                  _row_spec(BN, C), _row_spec(BN, C),
                  _p1_spec(BN, C, _NODE_GRID),
                  _w_spec(C), _w_spec(1), _w_spec(C), _w_spec(1),
                  _w_spec(C), _w_spec(C), _w_spec(1)],
        out_specs=[_row_spec(BN, C)] * 3,
        out_shape=_NODE_OUT3,
    )(eps.reshape(1, 1), gf, parts, parts, w1, b1.reshape(1, C),
      w2, b2.reshape(1, C), am, bm, b1n.reshape(1, C))


def _tc_mid(ug, vg, w2, b2):
    return pl.pallas_call(
        _mid_body,
        grid=(_EDGE_GRID,),
        in_specs=[_row_spec(BE, C), _row_spec(BE, C), _w_spec(C), _w_spec(1)],
        out_specs=_row_spec(BE, C),
        out_shape=jax.ShapeDtypeStruct((EP, C), _f32),
    )(ug, vg, w2, b2.reshape(1, C))


def _tc_post(parts, cnts, gp, am, bm, b1n):
    return pl.pallas_call(
        _post_body,
        grid=(_NODE_GRID,),
        in_specs=[_row_spec(BN, C), _p1_spec(BN, C, _NODE_GRID),
                  _row_spec(BN, C), _p1_spec(BN, C, _NODE_GRID),
                  _row_spec(BN, C),
                  _w_spec(C), _w_spec(C), _w_spec(1)],
        out_specs=[_row_spec(BN, C)] * 3,
        out_shape=_NODE_OUT3,
    )(parts, parts, cnts, cnts, gp, am, bm, b1n.reshape(1, C))


def _tc_fin(parts, cnts, gp, fw, fb):
    return pl.pallas_call(
        _fin_body,
        grid=(_NODE_GRID,),
        in_specs=[_row_spec(BN, C), _p1_spec(BN, C, _NODE_GRID),
                  _row_spec(BN, C), _p1_spec(BN, C, _NODE_GRID),
                  _row_spec(BN, C), _w_spec(C), _w_spec(1)],
        out_specs=_row_spec(BN, C),
        out_shape=jax.ShapeDtypeStruct((NP, C), _f32),
    )(parts, parts, cnts, cnts, gp, fw, fb)


def kernel(x, edge_index, edge_attr, batch, av_w, av_b, gin_eps,
           l1_w1, l1_b1, l1_w2, l1_b2,
           l2_w1, l2_b1, l2_w2, l2_b2,
           l3_w1, l3_b1, l3_w2, l3_b2,
           l4_w1, l4_b1, l4_w2, l4_b2,
           fc_w, fc_b):
    x2 = x.reshape(N, 2 * C)
    x2p = jnp.pad(x2, ((0, NP - N), (0, 0)))
    srcp = jnp.pad(edge_index[0], (0, EP - E), constant_values=N)
    dstp = jnp.pad(edge_index[1], (0, EP - E), constant_values=N)
    def split_w1(w1):
        return w1[:C] - w1[C:], w1[C:]

    am2, bm2 = split_w1(l2_w1)
    am3, bm3 = split_w1(l3_w1)
    am4, bm4 = split_w1(l4_w1)
    fw = jnp.zeros((C, C), _f32).at[:, :2].set(fc_w)
    fb = jnp.zeros((1, C), _f32).at[0, :2].set(fc_b)

    gf = _tc_fuse(x2p, av_w, av_b)
    cnts = _sc_cnt(dstp)
    parts0 = _sc_gin(gf, srcp, dstp)
    g1, u2, v2 = _tc_gin(gin_eps, gf, parts0, l1_w1, l1_b1, l1_w2, l1_b2,
                         am2, bm2, l2_b1)

    ug, vg = _sc_gather2(u2, v2, srcp, dstp)
    h2 = _tc_mid(ug, vg, l2_w2, l2_b2)
    parts2 = _sc_scatter(h2, dstp)
    g2, u3, v3 = _tc_post(parts2, cnts, g1, am3, bm3, l3_b1)

    ug, vg = _sc_gather2(u3, v3, srcp, dstp)
    h3 = _tc_mid(ug, vg, l3_w2, l3_b2)
    parts3 = _sc_scatter(h3, dstp)
    g3, u4, v4 = _tc_post(parts3, cnts, g2, am4, bm4, l4_b1)

    ug, vg = _sc_gather2(u4, v4, srcp, dstp)
    h4 = _tc_mid(ug, vg, l4_w2, l4_b2)
    parts4 = _sc_scatter(h4, dstp)
    outp = _tc_fin(parts4, cnts, g3, fw, fb)
    return outp[:N, :2]


# speedup vs baseline: 2.3439x; 1.1133x over previous
import functools

import jax
import jax.numpy as jnp
from jax import lax
from jax.experimental import pallas as pl
from jax.experimental.pallas import tpu as pltpu
from jax.experimental.pallas import tpu_sc as plsc

N = 10000
E = 160000
C = 128
NP = 10240
CH = 128
NWORK = 32
PER_TILE = 5120
NCHUNK = PER_TILE // CH
EP = NWORK * PER_TILE
RSLICE = NP // 16
SCALE = float(1.0 / (1.0 + 1e-5) ** 0.5)
BN = 1024
BE = 2048

_f32 = jnp.float32


def _wid_base():
    cid = lax.axis_index("c")
    sid = lax.axis_index("s")
    wid = cid * 16 + sid
    return cid, sid, wid * PER_TILE


def _gin_sc_body(gf, srcp, dstp, parts,
                 si_all, di0, di1, rows0, rows1, acc,
                 sG0, sG1, sI0, sI1):
    cid, sid, base0 = _wid_base()
    r0 = sid * RSLICE
    rows, dis = (rows0, rows1), (di0, di1)
    sGs, sIs = (sG0, sG1), (sI0, sI1)

    def fill(i, carry):
        zv = jnp.zeros((16,), _f32)
        for k in range(C // 16):
            rows0[i, pl.ds(k * 16, 16)] = zv
        return carry

    lax.fori_loop(0, CH, fill, 0)
    for k in range(RSLICE // CH):
        pltpu.sync_copy(rows0, acc.at[pl.ds(r0 + k * CH, CH)])
    plsc.subcore_barrier()
    pltpu.sync_copy(srcp.at[pl.ds(base0, PER_TILE)], si_all)

    def _gat(j, p):
        return pltpu.make_async_copy(
            gf.at[si_all.at[pl.ds(j * CH, CH)]], rows[p], sGs[p])

    def _idx(j, p):
        return pltpu.make_async_copy(
            dstp.at[pl.ds(base0 + j * CH, CH)], dis[p], sIs[p])

    _gat(0, 0).start()
    _idx(0, 0).start()

    def body(i, carry):
        _gat(2 * i + 1, 1).start()
        _idx(2 * i + 1, 1).start()
        _gat(2 * i, 0).wait()
        _idx(2 * i, 0).wait()
        pltpu.sync_copy(rows0, acc.at[di0], add=True)
        @pl.when(i < NCHUNK // 2 - 1)
        def _():
            _gat(2 * i + 2, 0).start()
            _idx(2 * i + 2, 0).start()
        _gat(2 * i + 1, 1).wait()
        _idx(2 * i + 1, 1).wait()
        pltpu.sync_copy(rows1, acc.at[di1], add=True)
        return carry

    lax.fori_loop(0, NCHUNK // 2, body, 0)
    plsc.subcore_barrier()
    out0 = cid * NP + r0
    for k in range(RSLICE // CH):
        pltpu.sync_copy(acc.at[pl.ds(r0 + k * CH, CH)], rows0)
        pltpu.sync_copy(rows0, parts.at[pl.ds(out0 + k * CH, CH)])


def _cnt_sc_body(dstp, cnts, di0, di1, ones, w16, cacc, sI0, sI1):
    cid, sid, base0 = _wid_base()
    r0 = sid * RSLICE
    dis, sIs = (di0, di1), (sI0, sI1)

    def fill(i, carry):
        ov = jnp.full((16,), 1.0, _f32)
        zv = jnp.zeros((16,), _f32)
        for k in range(C // 16):
            ones[i, pl.ds(k * 16, 16)] = ov
            w16[i, pl.ds(k * 16, 16)] = zv
        return carry

    lax.fori_loop(0, CH, fill, 0)
    for k in range(RSLICE // CH):
        pltpu.sync_copy(w16, cacc.at[pl.ds(r0 + k * CH, CH)])
    plsc.subcore_barrier()

    def _idx(j, p):
        return pltpu.make_async_copy(
            dstp.at[pl.ds(base0 + j * CH, CH)], dis[p], sIs[p])

    _idx(0, 0).start()

    def body(i, carry):
        _idx(2 * i + 1, 1).start()
        _idx(2 * i, 0).wait()
        pltpu.sync_copy(ones, cacc.at[di0], add=True)

        @pl.when(i < NCHUNK // 2 - 1)
        def _():
            _idx(2 * i + 2, 0).start()
        _idx(2 * i + 1, 1).wait()
        pltpu.sync_copy(ones, cacc.at[di1], add=True)
        return carry

    lax.fori_loop(0, NCHUNK // 2, body, 0)
    plsc.subcore_barrier()
    out0 = cid * NP + r0
    for k in range(RSLICE // CH):
        pltpu.sync_copy(cacc.at[pl.ds(r0 + k * CH, CH)], w16)
        pltpu.sync_copy(w16, cnts.at[pl.ds(out0 + k * CH, CH)])


def _gat_sc_body(u_hbm, v_hbm, srcp, dstp, ug, vg,
                 si_all, di_all, ub0, vb0, ub1, vb1, sU0, sV0, sU1, sV1):
    _, _, base0 = _wid_base()
    pltpu.sync_copy(srcp.at[pl.ds(base0, PER_TILE)], si_all)
    pltpu.sync_copy(dstp.at[pl.ds(base0, PER_TILE)], di_all)
    ubs, vbs = (ub0, ub1), (vb0, vb1)
    sUs, sVs = (sU0, sU1), (sV0, sV1)

    def _gat(j, p):
        off = pl.ds(j * CH, CH)
        return (pltpu.make_async_copy(u_hbm.at[di_all.at[off]], ubs[p], sUs[p]),
                pltpu.make_async_copy(v_hbm.at[si_all.at[off]], vbs[p], sVs[p]))

    def _put(j, p):
        hb = pl.ds(base0 + j * CH, CH)
        pltpu.sync_copy(ubs[p], ug.at[hb])
        pltpu.sync_copy(vbs[p], vg.at[hb])

    def start(cs):
        for c in cs:
            c.start()

    def wait(cs):
        for c in cs:
            c.wait()

    start(_gat(0, 0))

    def body(i, carry):
        @pl.when(i > 0)
        def _():
            start(_gat(2 * i, 0))
            wait(_gat(2 * i - 1, 1))
            _put(2 * i - 1, 1)

        start(_gat(2 * i + 1, 1))
        wait(_gat(2 * i, 0))
        _put(2 * i, 0)
        return carry

    lax.fori_loop(0, NCHUNK // 2, body, 0)
    wait(_gat(NCHUNK - 1, 1))
    _put(NCHUNK - 1, 1)


def _sct_sc_body(h2, dstp, parts, di0, di1, rows0, rows1, acc,
                 sR0, sR1, sI0, sI1):
    cid, sid, base0 = _wid_base()
    r0 = sid * RSLICE
    rows, dis = (rows0, rows1), (di0, di1)
    sRs, sIs = (sR0, sR1), (sI0, sI1)

    def fill(i, carry):
        zv = jnp.zeros((16,), _f32)
        for k in range(C // 16):
            rows0[i, pl.ds(k * 16, 16)] = zv
        return carry

    lax.fori_loop(0, CH, fill, 0)
    for k in range(RSLICE // CH):
        pltpu.sync_copy(rows0, acc.at[pl.ds(r0 + k * CH, CH)])
    plsc.subcore_barrier()

    def _rd(j, p):
        return pltpu.make_async_copy(
            h2.at[pl.ds(base0 + j * CH, CH)], rows[p], sRs[p])

    def _idx(j, p):
        return pltpu.make_async_copy(
            dstp.at[pl.ds(base0 + j * CH, CH)], dis[p], sIs[p])

    _rd(0, 0).start()
    _idx(0, 0).start()

    def body(i, carry):
        _rd(2 * i + 1, 1).start()
        _idx(2 * i + 1, 1).start()
        _rd(2 * i, 0).wait()
        _idx(2 * i, 0).wait()
        pltpu.sync_copy(rows0, acc.at[di0], add=True)

        @pl.when(i < NCHUNK // 2 - 1)
        def _():
            _rd(2 * i + 2, 0).start()
            _idx(2 * i + 2, 0).start()
        _rd(2 * i + 1, 1).wait()
        _idx(2 * i + 1, 1).wait()
        pltpu.sync_copy(rows1, acc.at[di1], add=True)
        return carry

    lax.fori_loop(0, NCHUNK // 2, body, 0)
    plsc.subcore_barrier()
    out0 = cid * NP + r0
    for k in range(RSLICE // CH):
        pltpu.sync_copy(acc.at[pl.ds(r0 + k * CH, CH)], rows0)
        pltpu.sync_copy(rows0, parts.at[pl.ds(out0 + k * CH, CH)])


@functools.lru_cache(maxsize=None)
def _sc_kernels():
    mesh = plsc.VectorSubcoreMesh(core_axis_name="c", subcore_axis_name="s")
    gin = pl.kernel(
        _gin_sc_body,
        mesh=mesh,
        out_type=jax.ShapeDtypeStruct((2 * NP, C), _f32),
        scratch_types=[pltpu.VMEM((PER_TILE,), jnp.int32),
                       pltpu.VMEM((CH,), jnp.int32),
                       pltpu.VMEM((CH,), jnp.int32),
                       pltpu.VMEM((CH, C), _f32), pltpu.VMEM((CH, C), _f32),
                       pltpu.VMEM_SHARED((NP, C), _f32),
                       pltpu.SemaphoreType.DMA, pltpu.SemaphoreType.DMA,
                       pltpu.SemaphoreType.DMA, pltpu.SemaphoreType.DMA],
    )
    cntk = pl.kernel(
        _cnt_sc_body,
        mesh=mesh,
        out_type=jax.ShapeDtypeStruct((2 * NP, C), _f32),
        scratch_types=[pltpu.VMEM((CH,), jnp.int32),
                       pltpu.VMEM((CH,), jnp.int32),
                       pltpu.VMEM((CH, C), _f32), pltpu.VMEM((CH, C), _f32),
                       pltpu.VMEM_SHARED((NP, C), _f32),
                       pltpu.SemaphoreType.DMA, pltpu.SemaphoreType.DMA],
    )
    gather2 = pl.kernel(
        _gat_sc_body,
        mesh=mesh,
        out_type=[jax.ShapeDtypeStruct((EP, C), _f32),
                  jax.ShapeDtypeStruct((EP, C), _f32)],
        scratch_types=[pltpu.VMEM((PER_TILE,), jnp.int32),
                       pltpu.VMEM((PER_TILE,), jnp.int32),
                       pltpu.VMEM((CH, C), _f32), pltpu.VMEM((CH, C), _f32),
                       pltpu.VMEM((CH, C), _f32), pltpu.VMEM((CH, C), _f32),
                       pltpu.SemaphoreType.DMA, pltpu.SemaphoreType.DMA,
                       pltpu.SemaphoreType.DMA, pltpu.SemaphoreType.DMA],
    )
    scatter = pl.kernel(
        _sct_sc_body,
        mesh=mesh,
        out_type=jax.ShapeDtypeStruct((2 * NP, C), _f32),
        scratch_types=[pltpu.VMEM((CH,), jnp.int32),
                       pltpu.VMEM((CH,), jnp.int32),
                       pltpu.VMEM((CH, C), _f32), pltpu.VMEM((CH, C), _f32),
                       pltpu.VMEM_SHARED((NP, C), _f32),
                       pltpu.SemaphoreType.DMA, pltpu.SemaphoreType.DMA,
                       pltpu.SemaphoreType.DMA, pltpu.SemaphoreType.DMA],
    )
    return gin, cntk, gather2, scatter


def _sc_gin(*a):
    return _sc_kernels()[0](*a)


def _sc_cnt(*a):
    return _sc_kernels()[1](*a)


def _sc_gather2(*a):
    return _sc_kernels()[2](*a)


def _sc_scatter(*a):
    return _sc_kernels()[3](*a)


def _dot(a, b):
    return jnp.dot(a, b, preferred_element_type=_f32)


def _fuse_body(x2, w, b, o):
    o[...] = _dot(x2[...], w[...]) + b[...]


def _gin_body(eps, gf, p0, p1, w1, b1, w2, b2, am, bm, b1n, g_o, u_o, v_o):
    z = gf[...] * (1.0 + eps[0, 0]) + p0[...] + p1[...]
    h = jnp.maximum((_dot(z, w1[...]) + b1[...]) * SCALE, 0.0)
    h = jnp.maximum((_dot(h, w2[...]) + b2[...]) * SCALE, 0.0)
    g = h * SCALE
    g_o[...] = g
    u_o[...] = (_dot(g, am[...]) + b1n[...]) * SCALE
    v_o[...] = _dot(g, bm[...]) * SCALE


def _mid_body(ug, vg, w2, b2, o):
    h1 = jnp.maximum(ug[...] + vg[...], 0.0)
    o[...] = jnp.maximum((_dot(h1, w2[...]) + b2[...]) * SCALE, 0.0)


def _post_body(p0, p1, c0, c1, gp, am, bm, b1n, g_o, u_o, v_o):
    cnt = jnp.maximum(c0[:, :1] + c1[:, :1], 1.0)
    g = jnp.maximum(((p0[...] + p1[...]) / cnt + gp[...]) * SCALE, 0.0)
    g_o[...] = g
    u_o[...] = (_dot(g, am[...]) + b1n[...]) * SCALE
    v_o[...] = _dot(g, bm[...]) * SCALE


def _fin_body(p0, p1, c0, c1, gp, fw, fb, o):
    cnt = jnp.maximum(c0[:, :1] + c1[:, :1], 1.0)
    g4 = (p0[...] + p1[...]) / cnt + gp[...]
    o[...] = _dot(g4, fw[...]) + fb[...]


def _row_spec(bn, cols):
    return pl.BlockSpec((bn, cols), lambda i: (i, 0))


def _p1_spec(bn, cols, off):
    return pl.BlockSpec((bn, cols), lambda i: (i + off, 0))


def _w_spec(r, c=C):
    return pl.BlockSpec((r, c), lambda i: (0, 0))


_NODE_GRID = NP // BN
_EDGE_GRID = EP // BE


def _tc_fuse(x2p, av_w, av_b):
    return pl.pallas_call(
        _fuse_body,
        grid=(_NODE_GRID,),
        in_specs=[_row_spec(BN, 2 * C), _w_spec(2 * C), _w_spec(1)],
        out_specs=_row_spec(BN, C),
        out_shape=jax.ShapeDtypeStruct((NP, C), _f32),
    )(x2p, av_w, av_b.reshape(1, C))


_NODE_OUT3 = [jax.ShapeDtypeStruct((NP, C), _f32)] * 3


def _tc_gin(eps, gf, parts, w1, b1, w2, b2, am, bm, b1n):
    return pl.pallas_call(
        _gin_body,
        grid=(_NODE_GRID,),
        in_specs=[pl.BlockSpec((1, 1), lambda i: (0, 0)),
                  _row_spec(BN, C), _row_spec(BN, C),
                  _p1_spec(BN, C, _NODE_GRID),
                  _w_spec(C), _w_spec(1), _w_spec(C), _w_spec(1),
                  _w_spec(C), _w_spec(C), _w_spec(1)],
        out_specs=[_row_spec(BN, C)] * 3,
        out_shape=_NODE_OUT3,
    )(eps.reshape(1, 1), gf, parts, parts, w1, b1.reshape(1, C),
      w2, b2.reshape(1, C), am, bm, b1n.reshape(1, C))


def _tc_mid(ug, vg, w2, b2):
    return pl.pallas_call(
        _mid_body,
        grid=(_EDGE_GRID,),
        in_specs=[_row_spec(BE, C), _row_spec(BE, C), _w_spec(C), _w_spec(1)],
        out_specs=_row_spec(BE, C),
        out_shape=jax.ShapeDtypeStruct((EP, C), _f32),
    )(ug, vg, w2, b2.reshape(1, C))


def _tc_post(parts, cnts, gp, am, bm, b1n):
    return pl.pallas_call(
        _post_body,
        grid=(_NODE_GRID,),
        in_specs=[_row_spec(BN, C), _p1_spec(BN, C, _NODE_GRID),
                  _row_spec(BN, C), _p1_spec(BN, C, _NODE_GRID),
                  _row_spec(BN, C),
                  _w_spec(C), _w_spec(C), _w_spec(1)],
        out_specs=[_row_spec(BN, C)] * 3,
        out_shape=_NODE_OUT3,
    )(parts, parts, cnts, cnts, gp, am, bm, b1n.reshape(1, C))


def _tc_fin(parts, cnts, gp, fw, fb):
    return pl.pallas_call(
        _fin_body,
        grid=(_NODE_GRID,),
        in_specs=[_row_spec(BN, C), _p1_spec(BN, C, _NODE_GRID),
                  _row_spec(BN, C), _p1_spec(BN, C, _NODE_GRID),
                  _row_spec(BN, C), _w_spec(C), _w_spec(1)],
        out_specs=_row_spec(BN, C),
        out_shape=jax.ShapeDtypeStruct((NP, C), _f32),
    )(parts, parts, cnts, cnts, gp, fw, fb)


def kernel(x, edge_index, edge_attr, batch, av_w, av_b, gin_eps,
           l1_w1, l1_b1, l1_w2, l1_b2,
           l2_w1, l2_b1, l2_w2, l2_b2,
           l3_w1, l3_b1, l3_w2, l3_b2,
           l4_w1, l4_b1, l4_w2, l4_b2,
           fc_w, fc_b):
    x2 = x.reshape(N, 2 * C)
    x2p = jnp.pad(x2, ((0, NP - N), (0, 0)))
    srcp = jnp.pad(edge_index[0], (0, EP - E), constant_values=N)
    dstp = jnp.pad(edge_index[1], (0, EP - E), constant_values=N)
    def split_w1(w1):
        return w1[:C] - w1[C:], w1[C:]

    am2, bm2 = split_w1(l2_w1)
    am3, bm3 = split_w1(l3_w1)
    am4, bm4 = split_w1(l4_w1)
    fw = jnp.zeros((C, C), _f32).at[:, :2].set(fc_w)
    fb = jnp.zeros((1, C), _f32).at[0, :2].set(fc_b)

    gf = _tc_fuse(x2p, av_w, av_b)
    cnts = _sc_cnt(dstp)
    parts0 = _sc_gin(gf, srcp, dstp)
    g1, u2, v2 = _tc_gin(gin_eps, gf, parts0, l1_w1, l1_b1, l1_w2, l1_b2,
                         am2, bm2, l2_b1)

    ug, vg = _sc_gather2(u2, v2, srcp, dstp)
    h2 = _tc_mid(ug, vg, l2_w2, l2_b2)
    parts2 = _sc_scatter(h2, dstp)
    g2, u3, v3 = _tc_post(parts2, cnts, g1, am3, bm3, l3_b1)

    ug, vg = _sc_gather2(u3, v3, srcp, dstp)
    h3 = _tc_mid(ug, vg, l3_w2, l3_b2)
    parts3 = _sc_scatter(h3, dstp)
    g3, u4, v4 = _tc_post(parts3, cnts, g2, am4, bm4, l4_b1)

    ug, vg = _sc_gather2(u4, v4, srcp, dstp)
    h4 = _tc_mid(ug, vg, l4_w2, l4_b2)
    parts4 = _sc_scatter(h4, dstp)
    outp = _tc_fin(parts4, cnts, g3, fw, fb)
    return outp[:N, :2]


# speedup vs baseline: 2.4593x; 1.0492x over previous
import functools

import jax
import jax.numpy as jnp
from jax import lax
from jax.experimental import pallas as pl
from jax.experimental.pallas import tpu as pltpu
from jax.experimental.pallas import tpu_sc as plsc

N = 10000
E = 160000
C = 128
NP = 10240
CH = 128
NWORK = 32
PER_TILE = 5120
NCHUNK = PER_TILE // CH
EP = NWORK * PER_TILE
RSLICE = NP // 16
NC0G, NC1G = 58, 22
NC0A, NC1A = 64, 16
EP_PAD = EP + NC0A * CH
SCALE = float(1.0 / (1.0 + 1e-5) ** 0.5)
BN = 1024
BE = 2048

_f32 = jnp.float32


def _wid_base():
    cid = lax.axis_index("c")
    sid = lax.axis_index("s")
    wid = cid * 16 + sid
    return cid, sid, wid * PER_TILE


def _split_base(nc0, nc1):
    cid = lax.axis_index("c")
    sid = lax.axis_index("s")
    nc = jnp.where(cid == 0, nc0, nc1)
    g0 = jnp.where(cid == 0, sid * nc0, 16 * nc0 + sid * nc1)
    return nc, g0 * CH


def _gin_sc_body(gf, srcp, dstp, parts,
                 si_all, di0, di1, rows0, rows1, acc,
                 sG0, sG1, sI0, sI1):
    cid, sid, _ = _wid_base()
    nc, base0 = _split_base(NC0A, NC1A)
    r0 = sid * RSLICE
    rows, dis = (rows0, rows1), (di0, di1)
    sGs, sIs = (sG0, sG1), (sI0, sI1)

    def fill(i, carry):
        zv = jnp.zeros((16,), _f32)
        for k in range(C // 16):
            rows0[i, pl.ds(k * 16, 16)] = zv
        return carry

    lax.fori_loop(0, CH, fill, 0)
    for k in range(RSLICE // CH):
        pltpu.sync_copy(rows0, acc.at[pl.ds(r0 + k * CH, CH)])
    plsc.subcore_barrier()
    pltpu.sync_copy(srcp.at[pl.ds(base0, NC0A * CH)], si_all)

    def _gat(j, p):
        return pltpu.make_async_copy(
            gf.at[si_all.at[pl.ds(j * CH, CH)]], rows[p], sGs[p])

    def _idx(j, p):
        return pltpu.make_async_copy(
            dstp.at[pl.ds(base0 + j * CH, CH)], dis[p], sIs[p])

    _gat(0, 0).start()
    _idx(0, 0).start()

    def body(i, carry):
        _gat(2 * i + 1, 1).start()
        _idx(2 * i + 1, 1).start()
        _gat(2 * i, 0).wait()
        _idx(2 * i, 0).wait()
        pltpu.sync_copy(rows0, acc.at[di0], add=True)
        @pl.when(i < nc // 2 - 1)
        def _():
            _gat(2 * i + 2, 0).start()
            _idx(2 * i + 2, 0).start()
        _gat(2 * i + 1, 1).wait()
        _idx(2 * i + 1, 1).wait()
        pltpu.sync_copy(rows1, acc.at[di1], add=True)
        return carry

    lax.fori_loop(0, nc // 2, body, 0)
    plsc.subcore_barrier()
    out0 = cid * NP + r0
    for k in range(RSLICE // CH):
        pltpu.sync_copy(acc.at[pl.ds(r0 + k * CH, CH)], rows0)
        pltpu.sync_copy(rows0, parts.at[pl.ds(out0 + k * CH, CH)])


def _cnt_sc_body(dstp, cnts, di0, di1, ones, w16, cacc, sI0, sI1):
    cid, sid, base0 = _wid_base()
    r0 = sid * RSLICE
    dis, sIs = (di0, di1), (sI0, sI1)

    def fill(i, carry):
        ov = jnp.full((16,), 1.0, _f32)
        zv = jnp.zeros((16,), _f32)
        for k in range(C // 16):
            ones[i, pl.ds(k * 16, 16)] = ov
            w16[i, pl.ds(k * 16, 16)] = zv
        return carry

    lax.fori_loop(0, CH, fill, 0)
    for k in range(RSLICE // CH):
        pltpu.sync_copy(w16, cacc.at[pl.ds(r0 + k * CH, CH)])
    plsc.subcore_barrier()

    def _idx(j, p):
        return pltpu.make_async_copy(
            dstp.at[pl.ds(base0 + j * CH, CH)], dis[p], sIs[p])

    _idx(0, 0).start()

    def body(i, carry):
        _idx(2 * i + 1, 1).start()
        _idx(2 * i, 0).wait()
        pltpu.sync_copy(ones, cacc.at[di0], add=True)

        @pl.when(i < NCHUNK // 2 - 1)
        def _():
            _idx(2 * i + 2, 0).start()
        _idx(2 * i + 1, 1).wait()
        pltpu.sync_copy(ones, cacc.at[di1], add=True)
        return carry

    lax.fori_loop(0, NCHUNK // 2, body, 0)
    plsc.subcore_barrier()
    out0 = cid * NP + r0
    for k in range(RSLICE // CH):
        pltpu.sync_copy(cacc.at[pl.ds(r0 + k * CH, CH)], w16)
        pltpu.sync_copy(w16, cnts.at[pl.ds(out0 + k * CH, CH)])


def _gat_sc_body(u_hbm, v_hbm, srcp, dstp, ug, vg,
                 si_all, di_all, ub0, vb0, ub1, vb1, sU0, sV0, sU1, sV1):
    nc, base0 = _split_base(NC0G, NC1G)
    pltpu.sync_copy(srcp.at[pl.ds(base0, NC0G * CH)], si_all)
    pltpu.sync_copy(dstp.at[pl.ds(base0, NC0G * CH)], di_all)
    ubs, vbs = (ub0, ub1), (vb0, vb1)
    sUs, sVs = (sU0, sU1), (sV0, sV1)

    def _gat(j, p):
        off = pl.ds(j * CH, CH)
        return (pltpu.make_async_copy(u_hbm.at[di_all.at[off]], ubs[p], sUs[p]),
                pltpu.make_async_copy(v_hbm.at[si_all.at[off]], vbs[p], sVs[p]))

    def _put(j, p):
        hb = pl.ds(base0 + j * CH, CH)
        pltpu.sync_copy(ubs[p], ug.at[hb])
        pltpu.sync_copy(vbs[p], vg.at[hb])

    def start(cs):
        for c in cs:
            c.start()

    def wait(cs):
        for c in cs:
            c.wait()

    start(_gat(0, 0))

    def body(i, carry):
        @pl.when(i > 0)
        def _():
            start(_gat(2 * i, 0))
            wait(_gat(2 * i - 1, 1))
            _put(2 * i - 1, 1)

        start(_gat(2 * i + 1, 1))
        wait(_gat(2 * i, 0))
        _put(2 * i, 0)
        return carry

    lax.fori_loop(0, nc // 2, body, 0)
    wait(_gat(nc - 1, 1))
    _put(nc - 1, 1)


def _sct_sc_body(h2, dstp, parts, di0, di1, rows0, rows1, acc,
                 sR0, sR1, sI0, sI1):
    cid, sid, base0 = _wid_base()
    r0 = sid * RSLICE
    rows, dis = (rows0, rows1), (di0, di1)
    sRs, sIs = (sR0, sR1), (sI0, sI1)

    def fill(i, carry):
        zv = jnp.zeros((16,), _f32)
        for k in range(C // 16):
            rows0[i, pl.ds(k * 16, 16)] = zv
        return carry

    lax.fori_loop(0, CH, fill, 0)
    for k in range(RSLICE // CH):
        pltpu.sync_copy(rows0, acc.at[pl.ds(r0 + k * CH, CH)])
    plsc.subcore_barrier()

    def _rd(j, p):
        return pltpu.make_async_copy(
            h2.at[pl.ds(base0 + j * CH, CH)], rows[p], sRs[p])

    def _idx(j, p):
        return pltpu.make_async_copy(
            dstp.at[pl.ds(base0 + j * CH, CH)], dis[p], sIs[p])

    _rd(0, 0).start()
    _idx(0, 0).start()

    def body(i, carry):
        _rd(2 * i + 1, 1).start()
        _idx(2 * i + 1, 1).start()
        _rd(2 * i, 0).wait()
        _idx(2 * i, 0).wait()
        pltpu.sync_copy(rows0, acc.at[di0], add=True)

        @pl.when(i < NCHUNK // 2 - 1)
        def _():
            _rd(2 * i + 2, 0).start()
            _idx(2 * i + 2, 0).start()
        _rd(2 * i + 1, 1).wait()
        _idx(2 * i + 1, 1).wait()
        pltpu.sync_copy(rows1, acc.at[di1], add=True)
        return carry

    lax.fori_loop(0, NCHUNK // 2, body, 0)
    plsc.subcore_barrier()
    out0 = cid * NP + r0
    for k in range(RSLICE // CH):
        pltpu.sync_copy(acc.at[pl.ds(r0 + k * CH, CH)], rows0)
        pltpu.sync_copy(rows0, parts.at[pl.ds(out0 + k * CH, CH)])


@functools.lru_cache(maxsize=None)
def _sc_kernels():
    mesh = plsc.VectorSubcoreMesh(core_axis_name="c", subcore_axis_name="s")
    gin = pl.kernel(
        _gin_sc_body,
        mesh=mesh,
        out_type=jax.ShapeDtypeStruct((2 * NP, C), _f32),
        scratch_types=[pltpu.VMEM((NC0A * CH,), jnp.int32),
                       pltpu.VMEM((CH,), jnp.int32),
                       pltpu.VMEM((CH,), jnp.int32),
                       pltpu.VMEM((CH, C), _f32), pltpu.VMEM((CH, C), _f32),
                       pltpu.VMEM_SHARED((NP, C), _f32),
                       pltpu.SemaphoreType.DMA, pltpu.SemaphoreType.DMA,
                       pltpu.SemaphoreType.DMA, pltpu.SemaphoreType.DMA],
    )
    cntk = pl.kernel(
        _cnt_sc_body,
        mesh=mesh,
        out_type=jax.ShapeDtypeStruct((2 * NP, C), _f32),
        scratch_types=[pltpu.VMEM((CH,), jnp.int32),
                       pltpu.VMEM((CH,), jnp.int32),
                       pltpu.VMEM((CH, C), _f32), pltpu.VMEM((CH, C), _f32),
                       pltpu.VMEM_SHARED((NP, C), _f32),
                       pltpu.SemaphoreType.DMA, pltpu.SemaphoreType.DMA],
    )
    gather2 = pl.kernel(
        _gat_sc_body,
        mesh=mesh,
        out_type=[jax.ShapeDtypeStruct((EP, C), _f32),
                  jax.ShapeDtypeStruct((EP, C), _f32)],
        scratch_types=[pltpu.VMEM((NC0G * CH,), jnp.int32),
                       pltpu.VMEM((NC0G * CH,), jnp.int32),
                       pltpu.VMEM((CH, C), _f32), pltpu.VMEM((CH, C), _f32),
                       pltpu.VMEM((CH, C), _f32), pltpu.VMEM((CH, C), _f32),
                       pltpu.SemaphoreType.DMA, pltpu.SemaphoreType.DMA,
                       pltpu.SemaphoreType.DMA, pltpu.SemaphoreType.DMA],
    )
    scatter = pl.kernel(
        _sct_sc_body,
        mesh=mesh,
        out_type=jax.ShapeDtypeStruct((2 * NP, C), _f32),
        scratch_types=[pltpu.VMEM((CH,), jnp.int32),
                       pltpu.VMEM((CH,), jnp.int32),
                       pltpu.VMEM((CH, C), _f32), pltpu.VMEM((CH, C), _f32),
                       pltpu.VMEM_SHARED((NP, C), _f32),
                       pltpu.SemaphoreType.DMA, pltpu.SemaphoreType.DMA,
                       pltpu.SemaphoreType.DMA, pltpu.SemaphoreType.DMA],
    )
    return gin, cntk, gather2, scatter


def _sc_gin(*a):
    return _sc_kernels()[0](*a)


def _sc_cnt(*a):
    return _sc_kernels()[1](*a)


def _sc_gather2(*a):
    return _sc_kernels()[2](*a)


def _sc_scatter(*a):
    return _sc_kernels()[3](*a)


def _dot(a, b):
    return jnp.dot(a, b, preferred_element_type=_f32)


def _fuse_body(x2, w, b, o):
    o[...] = _dot(x2[...], w[...]) + b[...]


def _gin_body(eps, gf, p0, p1, w1, b1, w2, b2, am, bm, b1n, g_o, u_o, v_o):
    z = gf[...] * (1.0 + eps[0, 0]) + p0[...] + p1[...]
    h = jnp.maximum((_dot(z, w1[...]) + b1[...]) * SCALE, 0.0)
    h = jnp.maximum((_dot(h, w2[...]) + b2[...]) * SCALE, 0.0)
    g = h * SCALE
    g_o[...] = g
    u_o[...] = (_dot(g, am[...]) + b1n[...]) * SCALE
    v_o[...] = _dot(g, bm[...]) * SCALE


def _mid_body(ug, vg, w2, b2, o):
    h1 = jnp.maximum(ug[...] + vg[...], 0.0)
    o[...] = jnp.maximum((_dot(h1, w2[...]) + b2[...]) * SCALE, 0.0)


def _post_body(p0, p1, c0, c1, gp, am, bm, b1n, g_o, u_o, v_o):
    cnt = jnp.maximum(c0[:, :1] + c1[:, :1], 1.0)
    g = jnp.maximum(((p0[...] + p1[...]) / cnt + gp[...]) * SCALE, 0.0)
    g_o[...] = g
    u_o[...] = (_dot(g, am[...]) + b1n[...]) * SCALE
    v_o[...] = _dot(g, bm[...]) * SCALE


def _fin_body(p0, p1, c0, c1, gp, fw, fb, o):
    cnt = jnp.maximum(c0[:, :1] + c1[:, :1], 1.0)
    g4 = (p0[...] + p1[...]) / cnt + gp[...]
    o[...] = _dot(g4, fw[...]) + fb[...]


def _row_spec(bn, cols):
    return pl.BlockSpec((bn, cols), lambda i: (i, 0))


def _p1_spec(bn, cols, off):
    return pl.BlockSpec((bn, cols), lambda i: (i + off, 0))


def _w_spec(r, c=C):
    return pl.BlockSpec((r, c), lambda i: (0, 0))


_NODE_GRID = NP // BN
_EDGE_GRID = EP // BE


def _tc_fuse(x2p, av_w, av_b):
    return pl.pallas_call(
        _fuse_body,
        grid=(_NODE_GRID,),
        in_specs=[_row_spec(BN, 2 * C), _w_spec(2 * C), _w_spec(1)],
        out_specs=_row_spec(BN, C),
        out_shape=jax.ShapeDtypeStruct((NP, C), _f32),
    )(x2p, av_w, av_b.reshape(1, C))


_NODE_OUT3 = [jax.ShapeDtypeStruct((NP, C), _f32)] * 3


def _tc_gin(eps, gf, parts, w1, b1, w2, b2, am, bm, b1n):
    return pl.pallas_call(
        _gin_body,
        grid=(_NODE_GRID,),
        in_specs=[pl.BlockSpec((1, 1), lambda i: (0, 0)),
                  _row_spec(BN, C), _row_spec(BN, C),
                  _p1_spec(BN, C, _NODE_GRID),
                  _w_spec(C), _w_spec(1), _w_spec(C), _w_spec(1),
                  _w_spec(C), _w_spec(C), _w_spec(1)],
        out_specs=[_row_spec(BN, C)] * 3,
        out_shape=_NODE_OUT3,
    )(eps.reshape(1, 1), gf, parts, parts, w1, b1.reshape(1, C),
      w2, b2.reshape(1, C), am, bm, b1n.reshape(1, C))


def _tc_mid(ug, vg, w2, b2):
    return pl.pallas_call(
        _mid_body,
        grid=(_EDGE_GRID,),
        in_specs=[_row_spec(BE, C), _row_spec(BE, C), _w_spec(C), _w_spec(1)],
        out_specs=_row_spec(BE, C),
        out_shape=jax.ShapeDtypeStruct((EP, C), _f32),
    )(ug, vg, w2, b2.reshape(1, C))


def _tc_post(parts, cnts, gp, am, bm, b1n):
    return pl.pallas_call(
        _post_body,
        grid=(_NODE_GRID,),
        in_specs=[_row_spec(BN, C), _p1_spec(BN, C, _NODE_GRID),
                  _row_spec(BN, C), _p1_spec(BN, C, _NODE_GRID),
                  _row_spec(BN, C),
                  _w_spec(C), _w_spec(C), _w_spec(1)],
        out_specs=[_row_spec(BN, C)] * 3,
        out_shape=_NODE_OUT3,
    )(parts, parts, cnts, cnts, gp, am, bm, b1n.reshape(1, C))


def _tc_fin(parts, cnts, gp, fw, fb):
    return pl.pallas_call(
        _fin_body,
        grid=(_NODE_GRID,),
        in_specs=[_row_spec(BN, C), _p1_spec(BN, C, _NODE_GRID),
                  _row_spec(BN, C), _p1_spec(BN, C, _NODE_GRID),
                  _row_spec(BN, C), _w_spec(C), _w_spec(1)],
        out_specs=_row_spec(BN, C),
        out_shape=jax.ShapeDtypeStruct((NP, C), _f32),
    )(parts, parts, cnts, cnts, gp, fw, fb)


def kernel(x, edge_index, edge_attr, batch, av_w, av_b, gin_eps,
           l1_w1, l1_b1, l1_w2, l1_b2,
           l2_w1, l2_b1, l2_w2, l2_b2,
           l3_w1, l3_b1, l3_w2, l3_b2,
           l4_w1, l4_b1, l4_w2, l4_b2,
           fc_w, fc_b):
    x2 = x.reshape(N, 2 * C)
    x2p = jnp.pad(x2, ((0, NP - N), (0, 0)))
    srcp = jnp.pad(edge_index[0], (0, EP_PAD - E), constant_values=N)
    dstp = jnp.pad(edge_index[1], (0, EP_PAD - E), constant_values=N)
    def split_w1(w1):
        return w1[:C] - w1[C:], w1[C:]

    am2, bm2 = split_w1(l2_w1)
    am3, bm3 = split_w1(l3_w1)
    am4, bm4 = split_w1(l4_w1)
    fw = jnp.zeros((C, C), _f32).at[:, :2].set(fc_w)
    fb = jnp.zeros((1, C), _f32).at[0, :2].set(fc_b)

    gf = _tc_fuse(x2p, av_w, av_b)
    cnts = _sc_cnt(dstp)
    parts0 = _sc_gin(gf, srcp, dstp)
    g1, u2, v2 = _tc_gin(gin_eps, gf, parts0, l1_w1, l1_b1, l1_w2, l1_b2,
                         am2, bm2, l2_b1)

    ug, vg = _sc_gather2(u2, v2, srcp, dstp)
    h2 = _tc_mid(ug, vg, l2_w2, l2_b2)
    parts2 = _sc_scatter(h2, dstp)
    g2, u3, v3 = _tc_post(parts2, cnts, g1, am3, bm3, l3_b1)

    ug, vg = _sc_gather2(u3, v3, srcp, dstp)
    h3 = _tc_mid(ug, vg, l3_w2, l3_b2)
    parts3 = _sc_scatter(h3, dstp)
    g3, u4, v4 = _tc_post(parts3, cnts, g2, am4, bm4, l4_b1)

    ug, vg = _sc_gather2(u4, v4, srcp, dstp)
    h4 = _tc_mid(ug, vg, l4_w2, l4_b2)
    parts4 = _sc_scatter(h4, dstp)
    outp = _tc_fin(parts4, cnts, g3, fw, fb)
    return outp[:N, :2]
